# baseline (speedup 1.0000x reference)
import jax
import jax.numpy as jnp
from jax.experimental import pallas as pl

PC_RANGE = jnp.array([-51.2, -51.2, -5.0, 51.2, 51.2, 3.0], dtype=jnp.float32)
VOXEL = jnp.array([0.2, 0.2, 8.0], dtype=jnp.float32)
BEV_W = 512
BEV_H = 512
SCALE_XY = BEV_W * BEV_H
SCALE_X = BEV_W
XY_OFFSET = VOXEL[:2] / 2.0 + PC_RANGE[:2]
N_RAW_FEAT = 5
N_BEV_FEAT = 64


def _mm_body(feats_ref, w_ref, out_ref):
    out_ref[...] = jnp.dot(feats_ref[...], w_ref[...].T,
                           preferred_element_type=jnp.float32)


def kernel(points, W, gamma, beta, batch_size):
    mask = jnp.all((points[:, 1:4] >= PC_RANGE[:3]) & (points[:, 1:4] < PC_RANGE[3:] - 0.001), axis=1)
    maskf = mask.astype(jnp.float32)
    N = points.shape[0]
    B_STATIC = 2
    sentinel = jnp.asarray(batch_size, dtype=jnp.int32) * SCALE_XY
    bev_coord = jnp.floor((points[:, 1:3] - PC_RANGE[:2]) / VOXEL[:2]).astype(jnp.int32)
    flat = points[:, 0].astype(jnp.int32) * SCALE_XY + bev_coord[:, 1] * SCALE_X + bev_coord[:, 0]
    flat = jnp.where(mask, flat, sentinel)
    pillars_flat, idx = jnp.unique(flat, return_inverse=True, size=N, fill_value=sentinel)
    idx = idx.reshape(-1)
    P = N
    counts = jax.ops.segment_sum(jnp.ones((N,), jnp.float32), idx, num_segments=P)
    safe_counts = jnp.maximum(counts, 1.0)
    pmean = jax.ops.segment_sum(points[:, 1:4], idx, num_segments=P) / safe_counts[:, None]
    f_mean = points[:, 1:4] - pmean[idx]
    f_center = points[:, 1:3] - (bev_coord.astype(jnp.float32) * VOXEL[:2] + XY_OFFSET)
    feats = jnp.concatenate([points[:, 1:1 + N_RAW_FEAT], f_mean, f_center], axis=1)

    BLK = 800
    h = pl.pallas_call(
        _mm_body,
        grid=(N // BLK,),
        in_specs=[pl.BlockSpec((BLK, 10), lambda i: (i, 0)),
                  pl.BlockSpec((N_BEV_FEAT, 10), lambda i: (0, 0))],
        out_specs=pl.BlockSpec((BLK, N_BEV_FEAT), lambda i: (i, 0)),
        out_shape=jax.ShapeDtypeStruct((N, N_BEV_FEAT), jnp.float32),
    )(feats, W)

    n_valid = jnp.sum(maskf)
    mu = jnp.sum(h * maskf[:, None], axis=0) / n_valid
    var = jnp.sum(((h - mu) ** 2) * maskf[:, None], axis=0) / n_valid
    h = (h - mu) / jnp.sqrt(var + 1e-3) * gamma + beta
    h = jax.nn.relu(h)
    pf = jax.ops.segment_max(h, idx, num_segments=P)
    bev = jnp.zeros((B_STATIC * SCALE_XY, N_BEV_FEAT), h.dtype).at[pillars_flat].set(pf, mode="drop")
    bev = bev.reshape(B_STATIC, BEV_H, BEV_W, N_BEV_FEAT).transpose(0, 3, 1, 2)
    return bev
